# Initial kernel scaffold; baseline (speedup 1.0000x reference)
import jax
import jax.numpy as jnp
from jax.experimental import pallas as pl


def kernel(x, edge_index, g, Wl0, bl0, Wr0, Wl1, bl1, Wr1, Wl2, bl2, Wr2):
    raise NotImplementedError("write your pallas kernel here")


# speedup vs baseline: 4.5528x; 4.5528x over previous
import functools

import jax
import jax.numpy as jnp
from jax import lax
from jax.experimental import pallas as pl
from jax.experimental.pallas import tpu as pltpu
from jax.experimental.pallas import tpu_sc as plsc

N = 10000
E = 320000
D = 128

NC = 2
NS = 16
NW = NC * NS
EPW = E // NW
CH = 80
NCH = EPW // CH
RPT = 632
NP = NS * RPT

_mesh = plsc.VectorSubcoreMesh(core_axis_name="c", subcore_axis_name="s")


@functools.partial(
    pl.kernel,
    out_type=jax.ShapeDtypeStruct((2 * NP, D), jnp.float32),
    mesh=_mesh,
    scratch_types=[
        pltpu.VMEM((CH,), jnp.int32),
        pltpu.VMEM((CH,), jnp.int32),
        pltpu.VMEM((CH, D), jnp.float32),
        pltpu.VMEM_SHARED((NP, D), jnp.float32),
        pltpu.SemaphoreType.DMA,
    ],
)
def _sc_aggregate(x_hbm, src_hbm, dst_hbm, zeros_hbm, out_hbm,
                  idx_s, idx_d, rows, acc, sem):
    cid = lax.axis_index("c")
    sid = lax.axis_index("s")
    wid = sid * NC + cid

    pltpu.sync_copy(zeros_hbm, acc.at[pl.ds(sid * RPT, RPT)])
    plsc.subcore_barrier()

    def body(i, carry):
        base = wid * EPW + i * CH
        pltpu.sync_copy(src_hbm.at[pl.ds(base, CH)], idx_s)
        pltpu.sync_copy(dst_hbm.at[pl.ds(base, CH)], idx_d)
        pltpu.async_copy(x_hbm.at[idx_s], rows, sem).wait()
        pltpu.sync_copy(rows, acc.at[idx_d], add=True)
        return carry

    lax.fori_loop(0, NCH, body, 0)
    plsc.subcore_barrier()

    pltpu.sync_copy(acc.at[pl.ds(sid * RPT, RPT)],
                    out_hbm.at[pl.ds(cid * NP + sid * RPT, RPT)])


@functools.partial(
    pl.kernel,
    out_type=jax.ShapeDtypeStruct((2 * N,), jnp.float32),
    mesh=_mesh,
    scratch_types=[
        pltpu.VMEM((CH,), jnp.int32),
        pltpu.VMEM((CH,), jnp.float32),
        pltpu.VMEM((N,), jnp.float32),
        pltpu.VMEM_SHARED((N,), jnp.float32),
    ],
)
def _sc_count(dst_hbm, zeros_hbm, out_hbm, idx_d, ones, stage, acc):
    cid = lax.axis_index("c")
    sid = lax.axis_index("s")
    wid = sid * NC + cid

    for j in range(CH // 16):
        ones[pl.ds(j * 16, 16)] = jnp.ones((16,), jnp.float32)

    @pl.when(sid == 0)
    def _():
        pltpu.sync_copy(zeros_hbm, stage)
        pltpu.sync_copy(stage, acc)
    plsc.subcore_barrier()

    def body(i, carry):
        base = wid * EPW + i * CH
        pltpu.sync_copy(dst_hbm.at[pl.ds(base, CH)], idx_d)
        pltpu.sync_copy(ones, acc.at[idx_d], add=True)
        return carry

    lax.fori_loop(0, NCH, body, 0)
    plsc.subcore_barrier()

    @pl.when(sid == 0)
    def _():
        pltpu.sync_copy(acc, stage)
        pltpu.sync_copy(stage, out_hbm.at[pl.ds(cid * N, N)])


BN = 1000


def _combine_body(act, p_ref, cnt_ref, h_ref, wl_ref, bl_ref, wr_ref, o_ref):
    s = p_ref[0] + p_ref[1]
    c = cnt_ref[0] + cnt_ref[1]
    mean = s / jnp.maximum(c, 1.0)
    a = lax.dot_general(mean, wl_ref[...], (((1,), (1,)), ((), ())),
                        preferred_element_type=jnp.float32)
    b = lax.dot_general(h_ref[...], wr_ref[...], (((1,), (1,)), ((), ())),
                        preferred_element_type=jnp.float32)
    o = a + b + bl_ref[...]
    if act == "relu":
        o = jnp.maximum(o, 0.0)
    else:
        o = jnp.tanh(o)
    o_ref[...] = o


def _make_combine(act):
    return pl.pallas_call(
        functools.partial(_combine_body, act),
        grid=(N // BN,),
        in_specs=[
            pl.BlockSpec((2, BN, D), lambda i: (0, i, 0)),
            pl.BlockSpec((2, BN, 1), lambda i: (0, i, 0)),
            pl.BlockSpec((BN, D), lambda i: (i, 0)),
            pl.BlockSpec((D, D), lambda i: (0, 0)),
            pl.BlockSpec((1, D), lambda i: (0, 0)),
            pl.BlockSpec((D, D), lambda i: (0, 0)),
        ],
        out_specs=pl.BlockSpec((BN, D), lambda i: (i, 0)),
        out_shape=jax.ShapeDtypeStruct((N, D), jnp.float32),
    )


_combine_relu = _make_combine("relu")
_combine_tanh = _make_combine("tanh")


def kernel(x, edge_index, g, Wl0, bl0, Wr0, Wl1, bl1, Wr1, Wl2, bl2, Wr2):
    src = edge_index[0]
    dst = edge_index[1]
    zeros_rows = jnp.zeros((RPT, D), jnp.float32)
    zeros_n = jnp.zeros((N,), jnp.float32)

    cnt = _sc_count(dst, zeros_n).reshape(2, N, 1)

    h = x
    layers = [(Wl0, bl0, Wr0, _combine_relu),
              (Wl1, bl1, Wr1, _combine_relu),
              (Wl2, bl2, Wr2, _combine_tanh)]
    for Wl, bl, Wr, combine in layers:
        p = _sc_aggregate(h, src, dst, zeros_rows).reshape(2, NP, D)
        h = combine(p, cnt, h, Wl, bl.reshape(1, D), Wr)
    return h


# speedup vs baseline: 9.7388x; 2.1391x over previous
import functools

import jax
import jax.numpy as jnp
from jax import lax
from jax.experimental import pallas as pl
from jax.experimental.pallas import tpu as pltpu
from jax.experimental.pallas import tpu_sc as plsc

N = 10000
E = 320000
D = 128

NC = 2
NS = 16
NW = NC * NS
EPW = E // NW
CH = 80
NCH = EPW // CH
NPAIR = (NCH - 1) // 2
RPT = 632
NP = NS * RPT

_mesh = plsc.VectorSubcoreMesh(core_axis_name="c", subcore_axis_name="s")


@functools.partial(
    pl.kernel,
    out_type=jax.ShapeDtypeStruct((2 * NP, D), jnp.float32),
    mesh=_mesh,
    scratch_types=[
        pltpu.VMEM((EPW,), jnp.int32),
        pltpu.VMEM((CH,), jnp.int32),
        pltpu.VMEM((CH,), jnp.int32),
        pltpu.VMEM((CH, D), jnp.float32),
        pltpu.VMEM((CH, D), jnp.float32),
        pltpu.VMEM_SHARED((NP, D), jnp.float32),
        pltpu.SemaphoreType.DMA,
        pltpu.SemaphoreType.DMA,
        pltpu.SemaphoreType.DMA,
        pltpu.SemaphoreType.DMA,
    ],
)
def _sc_aggregate(x_hbm, src_hbm, dst_hbm, zeros_hbm, out_hbm,
                  srcv, idx_d0, idx_d1, rows0, rows1, acc,
                  g0, g1, d0, d1):
    cid = lax.axis_index("c")
    sid = lax.axis_index("s")
    wid = sid * NC + cid
    ebase = wid * EPW

    pltpu.sync_copy(src_hbm.at[pl.ds(ebase, EPW)], srcv)
    pltpu.sync_copy(zeros_hbm, acc.at[pl.ds(sid * RPT, RPT)])
    plsc.subcore_barrier()

    def fetch_d(j, buf, sem):
        pltpu.async_copy(dst_hbm.at[pl.ds(ebase + j * CH, CH)], buf, sem)

    def wait_d(j, buf, sem):
        pltpu.make_async_copy(dst_hbm.at[pl.ds(ebase + j * CH, CH)],
                              buf, sem).wait()

    def issue_g(j, buf, sem):
        pltpu.async_copy(x_hbm.at[srcv.at[pl.ds(j * CH, CH)]], buf, sem)

    def wait_g(j, buf, sem):
        pltpu.make_async_copy(x_hbm.at[srcv.at[pl.ds(j * CH, CH)]],
                              buf, sem).wait()

    def scat(idx_buf, buf):
        pltpu.sync_copy(buf, acc.at[idx_buf], add=True)

    fetch_d(0, idx_d0, d0)
    issue_g(0, rows0, g0)

    def body(k, carry):
        j0 = 2 * k
        j1 = j0 + 1
        fetch_d(j1, idx_d1, d1)
        issue_g(j1, rows1, g1)
        wait_g(j0, rows0, g0)
        wait_d(j0, idx_d0, d0)
        scat(idx_d0, rows0)
        fetch_d(j0 + 2, idx_d0, d0)
        issue_g(j0 + 2, rows0, g0)
        wait_g(j1, rows1, g1)
        wait_d(j1, idx_d1, d1)
        scat(idx_d1, rows1)
        return carry

    lax.fori_loop(0, NPAIR, body, 0)
    wait_g(NCH - 1, rows0, g0)
    wait_d(NCH - 1, idx_d0, d0)
    scat(idx_d0, rows0)
    plsc.subcore_barrier()

    pltpu.sync_copy(acc.at[pl.ds(sid * RPT, RPT)],
                    out_hbm.at[pl.ds(cid * NP + sid * RPT, RPT)])


@functools.partial(
    pl.kernel,
    out_type=jax.ShapeDtypeStruct((2 * N,), jnp.float32),
    mesh=_mesh,
    scratch_types=[
        pltpu.VMEM((CH,), jnp.int32),
        pltpu.VMEM((CH,), jnp.float32),
        pltpu.VMEM((N,), jnp.float32),
        pltpu.VMEM_SHARED((N,), jnp.float32),
    ],
)
def _sc_count(dst_hbm, zeros_hbm, out_hbm, idx_d, ones, stage, acc):
    cid = lax.axis_index("c")
    sid = lax.axis_index("s")
    wid = sid * NC + cid

    for j in range(CH // 16):
        ones[pl.ds(j * 16, 16)] = jnp.ones((16,), jnp.float32)

    @pl.when(sid == 0)
    def _():
        pltpu.sync_copy(zeros_hbm, stage)
        pltpu.sync_copy(stage, acc)
    plsc.subcore_barrier()

    def body(i, carry):
        base = wid * EPW + i * CH
        pltpu.sync_copy(dst_hbm.at[pl.ds(base, CH)], idx_d)
        pltpu.sync_copy(ones, acc.at[idx_d], add=True)
        return carry

    lax.fori_loop(0, NCH, body, 0)
    plsc.subcore_barrier()

    @pl.when(sid == 0)
    def _():
        pltpu.sync_copy(acc, stage)
        pltpu.sync_copy(stage, out_hbm.at[pl.ds(cid * N, N)])


BN = 1000


def _combine_body(act, p_ref, cnt_ref, h_ref, wl_ref, bl_ref, wr_ref, o_ref):
    s = p_ref[0] + p_ref[1]
    c = cnt_ref[0] + cnt_ref[1]
    mean = s / jnp.maximum(c, 1.0)
    a = lax.dot_general(mean, wl_ref[...], (((1,), (1,)), ((), ())),
                        preferred_element_type=jnp.float32)
    b = lax.dot_general(h_ref[...], wr_ref[...], (((1,), (1,)), ((), ())),
                        preferred_element_type=jnp.float32)
    o = a + b + bl_ref[...]
    if act == "relu":
        o = jnp.maximum(o, 0.0)
    else:
        o = jnp.tanh(o)
    o_ref[...] = o


def _make_combine(act):
    return pl.pallas_call(
        functools.partial(_combine_body, act),
        grid=(N // BN,),
        in_specs=[
            pl.BlockSpec((2, BN, D), lambda i: (0, i, 0)),
            pl.BlockSpec((2, BN, 1), lambda i: (0, i, 0)),
            pl.BlockSpec((BN, D), lambda i: (i, 0)),
            pl.BlockSpec((D, D), lambda i: (0, 0)),
            pl.BlockSpec((1, D), lambda i: (0, 0)),
            pl.BlockSpec((D, D), lambda i: (0, 0)),
        ],
        out_specs=pl.BlockSpec((BN, D), lambda i: (i, 0)),
        out_shape=jax.ShapeDtypeStruct((N, D), jnp.float32),
    )


_combine_relu = _make_combine("relu")
_combine_tanh = _make_combine("tanh")


def kernel(x, edge_index, g, Wl0, bl0, Wr0, Wl1, bl1, Wr1, Wl2, bl2, Wr2):
    src = edge_index[0]
    dst = edge_index[1]
    zeros_rows = jnp.zeros((RPT, D), jnp.float32)
    zeros_n = jnp.zeros((N,), jnp.float32)

    cnt = _sc_count(dst, zeros_n).reshape(2, N, 1)

    h = x
    layers = [(Wl0, bl0, Wr0, _combine_relu),
              (Wl1, bl1, Wr1, _combine_relu),
              (Wl2, bl2, Wr2, _combine_tanh)]
    for Wl, bl, Wr, combine in layers:
        p = _sc_aggregate(h, src, dst, zeros_rows).reshape(2, NP, D)
        h = combine(p, cnt, h, Wl, bl.reshape(1, D), Wr)
    return h
